# baseline (speedup 1.0000x reference)
import jax
import jax.numpy as jnp
from jax.experimental import pallas as pl
from jax.experimental.pallas import tpu as pltpu

EMB = 300
K = 512
BLK = 2000
NROWS = 100000
NBLK = NROWS // BLK
NBUF = 5
NG = NBLK // NBUF


def _vq_chunk(eb, t, wb, wnb):
    rn = jnp.sum(eb * eb, axis=1, keepdims=True)
    mm = jax.lax.dot_general(
        eb.astype(jnp.bfloat16), wb, (((1,), (1,)), ((), ())),
        preferred_element_type=jnp.float32,
        precision=jax.lax.Precision.DEFAULT)
    wrow = jnp.where(t == 5, wnb[0:1], jnp.where(t == 6, wnb[1:2],
                     jnp.where(t == 7, wnb[2:3], wnb[3:4])))
    masked = (rn + wrow) - 2.0 * mm
    mins = jnp.min(masked, axis=1, keepdims=True)
    cols = jax.lax.broadcasted_iota(jnp.int32, (BLK, K), 1)
    enc = jnp.min(jnp.where(masked == mins, cols, K), axis=1, keepdims=True)
    onehot = (cols == enc).astype(jnp.bfloat16)
    qv = jax.lax.dot_general(
        onehot, wb, (((1,), (0,)), ((), ())),
        preferred_element_type=jnp.float32,
        precision=jax.lax.Precision.DEFAULT)
    return qv, mins


def _vq_pipe(x_hbm, e_hbm, wb_ref, wnb_ref, q_hbm, acc_ref, *scr):
    ebufs = scr[0:NBUF]
    xbufs = scr[NBUF:2 * NBUF]
    qbufs = scr[2 * NBUF:3 * NBUF]
    esems = scr[3 * NBUF:4 * NBUF]
    xsems = scr[4 * NBUF:5 * NBUF]
    qsems = scr[5 * NBUF:6 * NBUF]

    wb = wb_ref[...]
    wnb = wnb_ref[...]

    def in_copies(c, k):
        return (
            pltpu.make_async_copy(
                e_hbm.at[pl.ds(c * BLK, BLK), :], ebufs[k], esems[k]),
            pltpu.make_async_copy(
                x_hbm.at[pl.ds(c * BLK, BLK), :], xbufs[k], xsems[k]),
        )

    def q_copy(c, k):
        return pltpu.make_async_copy(
            qbufs[k], q_hbm.at[pl.ds(c * BLK, BLK), :], qsems[k])

    for k in range(NBUF):
        for cp in in_copies(k, k):
            cp.start()

    def group(g, total):
        for k in range(NBUF):
            c = g * NBUF + k
            for cp in in_copies(c, k):
                cp.wait()
            qv, mins = _vq_chunk(ebufs[k][...], xbufs[k][...][:, 0:1], wb, wnb)

            @pl.when(g > 0)
            def _drain():
                q_copy(c - NBUF, k).wait()

            qbufs[k][...] = qv
            q_copy(c, k).start()

            @pl.when(g < NG - 1)
            def _prefetch():
                for cp in in_copies(c + NBUF, k):
                    cp.start()

            total = total + jnp.sum(mins, axis=0, keepdims=True)
        return total

    total = jax.lax.fori_loop(0, NG, group, jnp.zeros((1, 1), jnp.float32))
    for k in range(NBUF):
        q_copy((NG - 1) * NBUF + k, k).wait()
    acc_ref[...] = total


def _wn_bias_table(W):
    wn = jnp.sum(W ** 2, axis=1)
    c = jnp.arange(K)
    inf = jnp.float32(jnp.inf)
    ranges = [(0, 377), (378, 433), (434, 488), (489, 511)]
    rows = [jnp.where((c >= lo) & (c < hi), wn, inf) for lo, hi in ranges]
    rows += [rows[-1]] * 4
    return jnp.stack(rows, axis=0)


def kernel(x, e, W):
    wnb = _wn_bias_table(W)
    wb = W.astype(jnp.bfloat16)
    q, acc = pl.pallas_call(
        _vq_pipe,
        in_specs=[
            pl.BlockSpec(memory_space=pl.ANY),
            pl.BlockSpec(memory_space=pl.ANY),
            pl.BlockSpec(memory_space=pltpu.MemorySpace.VMEM),
            pl.BlockSpec(memory_space=pltpu.MemorySpace.VMEM),
        ],
        out_specs=[
            pl.BlockSpec(memory_space=pl.ANY),
            pl.BlockSpec(memory_space=pltpu.MemorySpace.VMEM),
        ],
        out_shape=[
            jax.ShapeDtypeStruct((NROWS, EMB), jnp.float32),
            jax.ShapeDtypeStruct((1, 1), jnp.float32),
        ],
        scratch_shapes=(
            [pltpu.VMEM((BLK, EMB), jnp.float32)] * NBUF
            + [pltpu.VMEM((BLK, 8), jnp.int32)] * NBUF
            + [pltpu.VMEM((BLK, EMB), jnp.float32)] * NBUF
            + [pltpu.SemaphoreType.DMA] * (3 * NBUF)
        ),
        compiler_params=pltpu.CompilerParams(
            vmem_limit_bytes=100 * 1024 * 1024),
    )(x, e, wb, wnb)
    loss = 1.25 * acc[0, 0] / (NROWS * EMB)
    return q, loss


# speedup vs baseline: 1.0442x; 1.0442x over previous
import jax
import jax.numpy as jnp
from jax.experimental import pallas as pl

EMB = 300
K = 512
BLK = 5000
NROWS = 100000


def _vq_block(x_ref, e_ref, wb_ref, wnb_ref, q_ref, acc_ref):
    eb = e_ref[...]
    rn = jnp.sum(eb * eb, axis=1, keepdims=True)
    mm = jax.lax.dot_general(
        eb.astype(jnp.bfloat16), wb_ref[...], (((1,), (1,)), ((), ())),
        preferred_element_type=jnp.float32,
        precision=jax.lax.Precision.DEFAULT)

    t = x_ref[...][:, 0:1]
    wnb = wnb_ref[...]
    wrow = jnp.where(t == 5, wnb[0:1], jnp.where(t == 6, wnb[1:2],
                     jnp.where(t == 7, wnb[2:3], wnb[3:4])))
    masked = (rn + wrow) - 2.0 * mm
    mins = jnp.min(masked, axis=1, keepdims=True)
    cols = jax.lax.broadcasted_iota(jnp.int32, (BLK, K), 1)
    enc = jnp.min(jnp.where(masked == mins, cols, K), axis=1, keepdims=True)

    onehot = (cols == enc).astype(jnp.bfloat16)
    q_ref[...] = jax.lax.dot_general(
        onehot, wb_ref[...], (((1,), (0,)), ((), ())),
        preferred_element_type=jnp.float32,
        precision=jax.lax.Precision.DEFAULT)

    acc_ref[...] = jnp.sum(mins, axis=0, keepdims=True)[None, None]


def _wn_bias_table(W):
    wn = jnp.sum(W ** 2, axis=1)
    c = jnp.arange(K)
    inf = jnp.float32(jnp.inf)
    ranges = [(0, 377), (378, 433), (434, 488), (489, 511)]
    rows = [jnp.where((c >= lo) & (c < hi), wn, inf) for lo, hi in ranges]
    rows += [rows[-1]] * 4
    return jnp.stack(rows, axis=0)


def kernel(x, e, W):
    wnb = _wn_bias_table(W)
    wb = W.astype(jnp.bfloat16)
    grid = NROWS // BLK
    q, acc = pl.pallas_call(
        _vq_block,
        grid=(grid,),
        in_specs=[
            pl.BlockSpec((BLK, 8), lambda i: (i, 0)),
            pl.BlockSpec((BLK, EMB), lambda i: (i, 0)),
            pl.BlockSpec((K, EMB), lambda i: (0, 0)),
            pl.BlockSpec((8, K), lambda i: (0, 0)),
        ],
        out_specs=[
            pl.BlockSpec((BLK, EMB), lambda i: (i, 0)),
            pl.BlockSpec((1, 1, 1, 1), lambda i: (i, 0, 0, 0)),
        ],
        out_shape=[
            jax.ShapeDtypeStruct((NROWS, EMB), jnp.float32),
            jax.ShapeDtypeStruct((grid, 1, 1, 1), jnp.float32),
        ],
    )(x, e, wb, wnb)
    loss = 1.25 * jnp.sum(acc) / (NROWS * EMB)
    return q, loss


# speedup vs baseline: 1.0460x; 1.0017x over previous
import jax
import jax.numpy as jnp
from jax.experimental import pallas as pl

EMB = 300
K = 512
BLK = 4000
NROWS = 100000


def _vq_block(x_ref, e_ref, wb_ref, wnb_ref, q_ref, acc_ref):
    eb = e_ref[...]
    rn = jnp.sum(eb * eb, axis=1, keepdims=True)
    mm = jax.lax.dot_general(
        eb.astype(jnp.bfloat16), wb_ref[...], (((1,), (1,)), ((), ())),
        preferred_element_type=jnp.float32,
        precision=jax.lax.Precision.DEFAULT)

    t = x_ref[...][:, 0:1]
    wnb = wnb_ref[...]
    wrow = jnp.where(t == 5, wnb[0:1], jnp.where(t == 6, wnb[1:2],
                     jnp.where(t == 7, wnb[2:3], wnb[3:4])))
    masked = (rn + wrow) - 2.0 * mm
    mins = jnp.min(masked, axis=1, keepdims=True)
    cols = jax.lax.broadcasted_iota(jnp.int32, (BLK, K), 1)
    enc = jnp.min(jnp.where(masked == mins, cols, K), axis=1, keepdims=True)

    onehot = (cols == enc).astype(jnp.bfloat16)
    q_ref[...] = jax.lax.dot_general(
        onehot, wb_ref[...], (((1,), (0,)), ((), ())),
        preferred_element_type=jnp.float32,
        precision=jax.lax.Precision.DEFAULT)

    acc_ref[...] = jnp.sum(mins, axis=0, keepdims=True)[None, None]


def _wn_bias_table(W):
    wn = jnp.sum(W ** 2, axis=1)
    c = jnp.arange(K)
    inf = jnp.float32(jnp.inf)
    ranges = [(0, 377), (378, 433), (434, 488), (489, 511)]
    rows = [jnp.where((c >= lo) & (c < hi), wn, inf) for lo, hi in ranges]
    rows += [rows[-1]] * 4
    return jnp.stack(rows, axis=0)


def kernel(x, e, W):
    wnb = _wn_bias_table(W)
    wb = W.astype(jnp.bfloat16)
    grid = NROWS // BLK
    q, acc = pl.pallas_call(
        _vq_block,
        grid=(grid,),
        in_specs=[
            pl.BlockSpec((BLK, 8), lambda i: (i, 0)),
            pl.BlockSpec((BLK, EMB), lambda i: (i, 0)),
            pl.BlockSpec((K, EMB), lambda i: (0, 0)),
            pl.BlockSpec((8, K), lambda i: (0, 0)),
        ],
        out_specs=[
            pl.BlockSpec((BLK, EMB), lambda i: (i, 0)),
            pl.BlockSpec((1, 1, 1, 1), lambda i: (i, 0, 0, 0)),
        ],
        out_shape=[
            jax.ShapeDtypeStruct((NROWS, EMB), jnp.float32),
            jax.ShapeDtypeStruct((grid, 1, 1, 1), jnp.float32),
        ],
    )(x, e, wb, wnb)
    loss = 1.25 * jnp.sum(acc) / (NROWS * EMB)
    return q, loss
